# baseline (speedup 1.0000x reference)
import functools
import numpy as np
import jax
import jax.numpy as jnp
from jax import lax
from jax.experimental import pallas as pl
from jax.experimental.pallas import tpu as pltpu
from jax.experimental.pallas import tpu_sc as plsc


def _build_idx(n):
    h = n // 2
    rows, cols = np.triu_indices(n)
    flat = np.where(rows < h, rows * n + cols, (rows - h) * h + (cols - h))
    return flat.astype(np.int32)


def kernel(input):
    b, n, _ = input.shape
    h = n // 2
    m = (n * (n + 1)) // 2
    mtop = (h * (h + 1)) // 2 + h * h
    info = plsc.get_sparse_core_info()
    nw = info.num_cores * info.num_subcores
    bpw = b // nw
    idx_tab = jnp.asarray(_build_idx(n))
    mesh = plsc.VectorSubcoreMesh(core_axis_name="c", subcore_axis_name="s")

    lb_top, lb_bot = n.bit_length() - 1, h.bit_length() - 1

    @functools.partial(
        pl.kernel,
        mesh=mesh,
        out_type=jax.ShapeDtypeStruct((b, m), input.dtype),
        compiler_params=pltpu.CompilerParams(
            use_tc_tiling_on_sc=False, needs_layout_passes=False
        ),
        scratch_types=[
            pltpu.VMEM((h, n), jnp.float32),
            pltpu.VMEM((h, h), jnp.float32),
            pltpu.VMEM((m,), jnp.int32),
            pltpu.VMEM((m,), jnp.float32),
        ],
    )
    def k(in_hbm, idx_hbm, out_hbm, top_ref, bot_ref, idx_ref, out_ref):
        wid = lax.axis_index("s") * info.num_cores + lax.axis_index("c")
        pltpu.sync_copy(idx_hbm, idx_ref)

        def gather_one(ref, lb, v):
            f = idx_ref[pl.ds(16 * v, 16)]
            mask = jnp.int32((1 << lb) - 1)
            val = plsc.load_gather(ref, [lax.shift_right_logical(f, lb), f & mask])
            out_ref[pl.ds(16 * v, 16)] = val

        def per_batch(j, carry):
            bi = wid * bpw + j
            pltpu.sync_copy(in_hbm.at[bi, pl.ds(0, h), :], top_ref)
            pltpu.sync_copy(in_hbm.at[bi, pl.ds(h, h), pl.ds(h, h)], bot_ref)

            def top_body(v, c):
                gather_one(top_ref, lb_top, v)
                return c

            def bot_body(v, c):
                gather_one(bot_ref, lb_bot, v)
                return c

            lax.fori_loop(0, mtop // 16, top_body, 0)
            lax.fori_loop(mtop // 16, m // 16, bot_body, 0)
            pltpu.sync_copy(out_ref, out_hbm.at[bi])
            return carry

        lax.fori_loop(0, bpw, per_batch, 0)

    return k(input, idx_tab)


# speedup vs baseline: 1.5762x; 1.5762x over previous
import functools
import numpy as np
import jax
import jax.numpy as jnp
from jax import lax
from jax.experimental import pallas as pl
from jax.experimental.pallas import tpu as pltpu
from jax.experimental.pallas import tpu_sc as plsc


def _build_idx(n):
    h = n // 2
    rows, cols = np.triu_indices(n)
    flat = np.where(rows < h, rows * n + cols, (rows - h) * h + (cols - h))
    return flat.astype(np.int32)


def kernel(input):
    b, n, _ = input.shape
    h = n // 2
    m = (n * (n + 1)) // 2
    mtop = (h * (h + 1)) // 2 + h * h
    info = plsc.get_sparse_core_info()
    nw = info.num_cores * info.num_subcores
    bpw = b // nw
    idx_tab = jnp.asarray(_build_idx(n))
    mesh = plsc.VectorSubcoreMesh(core_axis_name="c", subcore_axis_name="s")

    lb_top, lb_bot = n.bit_length() - 1, h.bit_length() - 1

    @functools.partial(
        pl.kernel,
        mesh=mesh,
        out_type=jax.ShapeDtypeStruct((b, m), input.dtype),
        compiler_params=pltpu.CompilerParams(
            use_tc_tiling_on_sc=False, needs_layout_passes=False
        ),
        scratch_types=[
            pltpu.VMEM((h, n), jnp.float32),
            pltpu.VMEM((h, h), jnp.float32),
            pltpu.VMEM((m,), jnp.int32),
            pltpu.VMEM((m,), jnp.float32),
        ],
    )
    def k(in_hbm, idx_hbm, out_hbm, top_ref, bot_ref, idx_ref, out_ref):
        wid = lax.axis_index("s") * info.num_cores + lax.axis_index("c")
        pltpu.sync_copy(idx_hbm, idx_ref)

        def gather_one(ref, lb, v):
            f = idx_ref[pl.ds(16 * v, 16)]
            mask = jnp.int32((1 << lb) - 1)
            val = plsc.load_gather(ref, [lax.shift_right_logical(f, lb), f & mask])
            out_ref[pl.ds(16 * v, 16)] = val

        def per_batch(j, carry):
            bi = wid * bpw + j
            pltpu.sync_copy(in_hbm.at[bi, pl.ds(0, h), :], top_ref)
            pltpu.sync_copy(in_hbm.at[bi, pl.ds(h, h), pl.ds(h, h)], bot_ref)

            @plsc.parallel_loop(0, mtop // 16, unroll=8)
            def top_body(v):
                gather_one(top_ref, lb_top, v)

            @plsc.parallel_loop(mtop // 16, m // 16, unroll=8)
            def bot_body(v):
                gather_one(bot_ref, lb_bot, v)
            pltpu.sync_copy(out_ref, out_hbm.at[bi])
            return carry

        lax.fori_loop(0, bpw, per_batch, 0)

    return k(input, idx_tab)


# speedup vs baseline: 1.7796x; 1.1291x over previous
import functools
import jax
import jax.numpy as jnp
from jax import lax
from jax.experimental import pallas as pl
from jax.experimental.pallas import tpu as pltpu
from jax.experimental.pallas import tpu_sc as plsc


def kernel(input):
    b, n, _ = input.shape
    m = (n * (n + 1)) // 2
    n4 = n // 4
    info = plsc.get_sparse_core_info()
    nw = info.num_cores * info.num_subcores
    bpw = b // nw
    mesh = plsc.VectorSubcoreMesh(core_axis_name="c", subcore_axis_name="s")

    cstart = [g * n4 for g in range(4)]
    widths = [n - c for c in cstart]
    def off(r):
        return n * r - (r * (r - 1)) // 2

    chunk_lo = [off(g * n4) for g in range(4)] + [m]

    @functools.partial(
        pl.kernel,
        mesh=mesh,
        out_type=jax.ShapeDtypeStruct((b, m), input.dtype),
        compiler_params=pltpu.CompilerParams(
            use_tc_tiling_on_sc=False, needs_layout_passes=False
        ),
        scratch_types=[
            pltpu.VMEM((n4, widths[0]), jnp.float32),
            pltpu.VMEM((n4, widths[1]), jnp.float32),
            pltpu.VMEM((n4, widths[2]), jnp.float32),
            pltpu.VMEM((n4, widths[3]), jnp.float32),
            pltpu.VMEM((m,), jnp.float32),
            pltpu.SemaphoreType.DMA,
            pltpu.SemaphoreType.DMA,
        ],
    )
    def k(in_hbm, out_hbm, p0, p1, p2, p3, out_ref, in_sem, out_sem):
        pieces = [p0, p1, p2, p3]
        wid = lax.axis_index("s") * info.num_cores + lax.axis_index("c")
        b0 = wid * bpw

        def in_slice(bi, g):
            return in_hbm.at[bi, pl.ds(cstart[g], n4), pl.ds(cstart[g], widths[g])]

        def out_chunk(g):
            return out_ref.at[pl.ds(chunk_lo[g], chunk_lo[g + 1] - chunk_lo[g])]

        def out_hbm_chunk(bi, g):
            return out_hbm.at[bi, pl.ds(chunk_lo[g], chunk_lo[g + 1] - chunk_lo[g])]

        def pack_piece(g):
            ref = pieces[g]
            w = widths[g]
            for rl in range(n4):
                r = cstart[g] + rl
                L = n - r
                o = off(r)
                diag = r - cstart[g]
                nfull, rem = L // 16, L % 16
                for t in range(nfull):
                    out_ref[pl.ds(o + 16 * t, 16)] = ref[rl, pl.ds(diag + 16 * t, 16)]
                if rem and nfull:
                    out_ref[pl.ds(o + L - 16, 16)] = ref[rl, pl.ds(diag + L - 16, 16)]
                elif rem:
                    iota = jax.lax.iota(jnp.int32, 16)
                    v = ref[rl, pl.ds(w - 16, 16)]
                    idx = iota + jnp.int32(o + (n - 16) - r)
                    mask = iota >= jnp.int32(r - (n - 16))
                    plsc.store_scatter(out_ref, [idx], v, mask=mask)

        for g in range(4):
            pltpu.async_copy(in_slice(b0, g), pieces[g], in_sem)

        def per_batch(j, carry):
            bi = b0 + j
            for g in range(4):
                pltpu.make_async_copy(in_slice(bi, g), pieces[g], in_sem).wait()
            for g in range(4):
                @pl.when(j > 0)
                def _wait_out():
                    pltpu.make_async_copy(out_chunk(g), out_hbm_chunk(bi, g), out_sem).wait()

                pack_piece(g)

                @pl.when(j + 1 < bpw)
                def _refill():
                    pltpu.async_copy(in_slice(bi + 1, g), pieces[g], in_sem)

                pltpu.async_copy(out_chunk(g), out_hbm_chunk(bi, g), out_sem)
            return carry

        lax.fori_loop(0, bpw, per_batch, 0)
        for g in range(4):
            pltpu.make_async_copy(out_chunk(g), out_hbm_chunk(b0 + bpw - 1, g), out_sem).wait()

    return k(input)


# speedup vs baseline: 7.6289x; 4.2868x over previous
import functools
import jax
import jax.numpy as jnp
from jax.experimental import pallas as pl


def _off(n, r):
    return n * r - (r * (r - 1)) // 2


def _body_split(n, top_ref, bot_ref, out_ref):
    h = n // 2
    for r in range(h):
        out_ref[:, pl.ds(_off(n, r), n - r)] = top_ref[:, r, pl.ds(r, n - r)]
    for r in range(h, n):
        out_ref[:, pl.ds(_off(n, r), n - r)] = bot_ref[:, r - h, pl.ds(r - h, n - r)]


def _body_single(n, in_ref, out_ref):
    for r in range(n):
        out_ref[:, pl.ds(_off(n, r), n - r)] = in_ref[:, r, pl.ds(r, n - r)]


def kernel(input):
    b, n, _ = input.shape
    m = (n * (n + 1)) // 2
    h = n // 2
    bb = 64 if b % 64 == 0 else 1
    grid = (b // bb,)
    out_shape = jax.ShapeDtypeStruct((b, m), input.dtype)
    if h % 128 == 0:
        return pl.pallas_call(
            functools.partial(_body_split, n),
            grid=grid,
            in_specs=[
                pl.BlockSpec((bb, h, n), lambda i: (i, 0, 0)),
                pl.BlockSpec((bb, h, h), lambda i: (i, 1, 1)),
            ],
            out_specs=pl.BlockSpec((bb, m), lambda i: (i, 0)),
            out_shape=out_shape,
        )(input, input)
    return pl.pallas_call(
        functools.partial(_body_single, n),
        grid=grid,
        in_specs=[pl.BlockSpec((bb, n, n), lambda i: (i, 0, 0))],
        out_specs=pl.BlockSpec((bb, m), lambda i: (i, 0)),
        out_shape=out_shape,
    )(input)
